# baseline (device time: 191496 ns/iter reference)
import jax
import jax.numpy as jnp
from jax import lax
from jax.experimental import pallas as pl
from jax.experimental.pallas import tpu as pltpu

N_DEV = 8
SQ = 256
SKV = 4096
HQ = 8
DH = 128
D = HQ * DH
SCALE = 0.08838834764831843
NEG = -1e9
M_INIT = -1e30


def _body(x_ref, wq_ref, k_ref, v_ref, wo_ref, out_ref,
          q_buf, acc_buf, st_buf, bias_ref, pv_buf, loc_ref,
          sq_sem, rq_sem, sa_sem, ra_sem, ssem, rsem):
    d = lax.axis_index("i")
    left = lax.rem(d + N_DEV - 1, N_DEV)
    right = lax.rem(d + 1, N_DEV)

    def q_rdma(slot):
        return pltpu.make_async_remote_copy(
            src_ref=q_buf.at[slot], dst_ref=q_buf.at[slot + 1],
            send_sem=sq_sem.at[slot], recv_sem=rq_sem.at[slot + 1],
            device_id=(right,), device_id_type=pl.DeviceIdType.MESH)

    def acc_rdma(slot):
        return pltpu.make_async_remote_copy(
            src_ref=acc_buf.at[slot], dst_ref=acc_buf.at[slot + 1],
            send_sem=sa_sem.at[slot], recv_sem=ra_sem.at[slot + 1],
            device_id=(right,), device_id_type=pl.DeviceIdType.MESH)

    def st_rdma(slot):
        return pltpu.make_async_remote_copy(
            src_ref=st_buf.at[slot], dst_ref=st_buf.at[slot + 1],
            send_sem=ssem.at[slot], recv_sem=rsem.at[slot + 1],
            device_id=(right,), device_id_type=pl.DeviceIdType.MESH)

    barrier_sem = pltpu.get_barrier_semaphore()
    for nbr in (left, right):
        pl.semaphore_signal(barrier_sem, inc=1, device_id=(nbr,),
                            device_id_type=pl.DeviceIdType.MESH)
    pl.semaphore_wait(barrier_sem, 2)

    q = jnp.dot(x_ref[...], wq_ref[...], preferred_element_type=jnp.float32)
    q_buf[0] = (q * SCALE).astype(jnp.bfloat16)
    acc_buf[0] = jnp.zeros((SQ, D), jnp.bfloat16)
    st_buf[0, 0] = jnp.full((SQ, HQ), M_INIT, jnp.float32)
    st_buf[0, 1] = jnp.zeros((SQ, HQ), jnp.float32)

    def hop_body(hop, carry):
        @pl.when(hop > 0)
        def _():
            q_rdma(hop - 1).wait_recv()

        @pl.when(hop < N_DEV - 1)
        def _():
            q_rdma(hop).start()

        owner = lax.rem(d - hop + N_DEV, N_DEV)
        ri = lax.broadcasted_iota(jnp.int32, (SQ, SKV), 0)
        ci = lax.broadcasted_iota(jnp.int32, (SQ, SKV), 1)
        qb = owner * (SQ // 64) + ri // 64
        kb = d * (SKV // 64) + ci // 64
        keep = (qb == kb) | (kb == 0) | (lax.rem(qb + kb, 3) == 0)
        bias_ref[...] = jnp.where(keep, 0.0, NEG).astype(jnp.bfloat16)

        for h in range(HQ):
            qh = q_buf[hop, :, h * DH:(h + 1) * DH]
            s = lax.dot_general(
                qh, k_ref[h], (((1,), (1,)), ((), ())),
                preferred_element_type=jnp.float32)
            s = s + bias_ref[...]
            m_loc = jnp.max(s, axis=1, keepdims=True)
            p = jnp.exp(s - m_loc)
            l_loc = jnp.sum(p, axis=1, keepdims=True)
            pv = lax.dot_general(
                p.astype(jnp.bfloat16), v_ref[h], (((1,), (0,)), ((), ())),
                preferred_element_type=jnp.float32)
            pv_buf[:, h * DH:(h + 1) * DH] = pv
            loc_ref[0, :, h:h + 1] = m_loc
            loc_ref[1, :, h:h + 1] = l_loc

        @pl.when(hop > 0)
        def _():
            acc_rdma(hop - 1).wait_send()
            st_rdma(hop - 1).wait_send()

            @pl.when(hop - 1 < N_DEV - 1)
            def _():
                q_rdma(hop - 1).wait_send()

        @pl.when(hop > 0)
        def _():
            acc_rdma(hop - 1).wait_recv()
            st_rdma(hop - 1).wait_recv()

        for h in range(HQ):
            m_prev = st_buf[hop, 0, :, h:h + 1]
            l_prev = st_buf[hop, 1, :, h:h + 1]
            m_loc = loc_ref[0, :, h:h + 1]
            l_loc = loc_ref[1, :, h:h + 1]
            m_new = jnp.maximum(m_prev, m_loc)
            alpha = jnp.exp(m_prev - m_new)
            beta = jnp.exp(m_loc - m_new)
            acc = acc_buf[hop, :, h * DH:(h + 1) * DH].astype(jnp.float32)
            pv = pv_buf[:, h * DH:(h + 1) * DH]
            acc_buf[hop, :, h * DH:(h + 1) * DH] = (
                acc * alpha + pv * beta).astype(jnp.bfloat16)
            st_buf[hop, 0, :, h:h + 1] = m_new
            st_buf[hop, 1, :, h:h + 1] = l_prev * alpha + l_loc * beta

        acc_rdma(hop).start()
        st_rdma(hop).start()
        return carry

    lax.fori_loop(0, N_DEV, hop_body, 0)

    acc_rdma(N_DEV - 1).wait_send()
    st_rdma(N_DEV - 1).wait_send()
    acc_rdma(N_DEV - 1).wait_recv()
    st_rdma(N_DEV - 1).wait_recv()

    parts = []
    for h in range(HQ):
        acc = acc_buf[N_DEV, :, h * DH:(h + 1) * DH].astype(jnp.float32)
        l = st_buf[N_DEV, 1, :, h:h + 1]
        parts.append(acc / l)
    ctx = jnp.concatenate(parts, axis=1)
    out_ref[...] = jnp.dot(ctx.astype(jnp.bfloat16), wo_ref[...],
                           preferred_element_type=jnp.float32)


def kernel(x, Wq, K_ext, V_ext, Wo):
    xb = x[0].astype(jnp.bfloat16)
    wq = Wq.astype(jnp.bfloat16)
    wo = Wo.astype(jnp.bfloat16)
    kb = jnp.transpose(K_ext[0].astype(jnp.bfloat16), (1, 0, 2))
    vb = jnp.transpose(V_ext[0].astype(jnp.bfloat16), (1, 0, 2))

    out = pl.pallas_call(
        _body,
        out_shape=jax.ShapeDtypeStruct((SQ, D), jnp.float32),
        in_specs=[pl.BlockSpec(memory_space=pltpu.VMEM)] * 5,
        out_specs=pl.BlockSpec(memory_space=pltpu.VMEM),
        scratch_shapes=[
            pltpu.VMEM((N_DEV + 1, SQ, D), jnp.bfloat16),
            pltpu.VMEM((N_DEV + 1, SQ, D), jnp.bfloat16),
            pltpu.VMEM((N_DEV + 1, 2, SQ, HQ), jnp.float32),
            pltpu.VMEM((SQ, SKV), jnp.bfloat16),
            pltpu.VMEM((SQ, D), jnp.float32),
            pltpu.VMEM((2, SQ, HQ), jnp.float32),
            pltpu.SemaphoreType.DMA((N_DEV + 1,)),
            pltpu.SemaphoreType.DMA((N_DEV + 1,)),
            pltpu.SemaphoreType.DMA((N_DEV + 1,)),
            pltpu.SemaphoreType.DMA((N_DEV + 1,)),
            pltpu.SemaphoreType.DMA((N_DEV + 1,)),
            pltpu.SemaphoreType.DMA((N_DEV + 1,)),
        ],
        compiler_params=pltpu.CompilerParams(
            collective_id=0,
            vmem_limit_bytes=100 * 1024 * 1024,
        ),
    )(xb, wq, kb, vb, wo)
    return out[None]


# device time: 188731 ns/iter; 1.0147x vs baseline; 1.0147x over previous
import jax
import jax.numpy as jnp
from jax import lax
from jax.experimental import pallas as pl
from jax.experimental.pallas import tpu as pltpu

N_DEV = 8
SQ = 256
SKV = 4096
HQ = 8
DH = 128
D = HQ * DH
SCALE = 0.08838834764831843
NEG = -1e9
M_INIT = -1e30


def _body(x_ref, wq_ref, k_ref, v_ref, wo_ref, out_ref,
          q_buf, acc_buf, st_buf, bias_ref, pv_buf, loc_ref,
          sq_sem, rq_sem, sa_sem, ra_sem, ssem, rsem):
    d = lax.axis_index("i")
    left = lax.rem(d + N_DEV - 1, N_DEV)
    right = lax.rem(d + 1, N_DEV)

    def q_rdma(slot):
        return pltpu.make_async_remote_copy(
            src_ref=q_buf.at[slot], dst_ref=q_buf.at[slot + 1],
            send_sem=sq_sem.at[slot], recv_sem=rq_sem.at[slot + 1],
            device_id=(right,), device_id_type=pl.DeviceIdType.MESH)

    def acc_rdma(slot):
        return pltpu.make_async_remote_copy(
            src_ref=acc_buf.at[slot], dst_ref=acc_buf.at[slot + 1],
            send_sem=sa_sem.at[slot], recv_sem=ra_sem.at[slot + 1],
            device_id=(right,), device_id_type=pl.DeviceIdType.MESH)

    def st_rdma(slot):
        return pltpu.make_async_remote_copy(
            src_ref=st_buf.at[slot], dst_ref=st_buf.at[slot + 1],
            send_sem=ssem.at[slot], recv_sem=rsem.at[slot + 1],
            device_id=(right,), device_id_type=pl.DeviceIdType.MESH)

    barrier_sem = pltpu.get_barrier_semaphore()
    for nbr in (left, right):
        pl.semaphore_signal(barrier_sem, inc=1, device_id=(nbr,),
                            device_id_type=pl.DeviceIdType.MESH)
    pl.semaphore_wait(barrier_sem, 2)

    q = jnp.dot(x_ref[...], wq_ref[...], preferred_element_type=jnp.float32)
    q_buf[0] = (q * SCALE).astype(jnp.bfloat16)
    acc_buf[0] = jnp.zeros((SQ, D), jnp.bfloat16)
    st_buf[0, 0] = jnp.full((SQ, HQ), M_INIT, jnp.float32)
    st_buf[0, 1] = jnp.zeros((SQ, HQ), jnp.float32)

    def hop_body(hop, carry):
        @pl.when(hop > 0)
        def _():
            q_rdma(hop - 1).wait_recv()

        @pl.when(hop < N_DEV - 1)
        def _():
            q_rdma(hop).start()

        owner = lax.rem(d - hop + N_DEV, N_DEV)
        ri = lax.broadcasted_iota(jnp.int32, (SQ, SKV), 0)
        ci = lax.broadcasted_iota(jnp.int32, (SQ, SKV), 1)
        qb = owner * (SQ // 64) + ri // 64
        kb = d * (SKV // 64) + ci // 64
        keep = (qb == kb) | (kb == 0) | (lax.rem(qb + kb, 3) == 0)
        bias_ref[...] = jnp.where(keep, 0.0, NEG).astype(jnp.float32)

        for h in range(HQ):
            qh = q_buf[hop, :, h * DH:(h + 1) * DH]
            s = lax.dot_general(
                qh, k_ref[h], (((1,), (1,)), ((), ())),
                preferred_element_type=jnp.float32)
            s = s + bias_ref[...]
            m_loc = jnp.max(s, axis=1, keepdims=True)
            p = jnp.exp(s - m_loc)
            l_loc = jnp.sum(p, axis=1, keepdims=True)
            pv = lax.dot_general(
                p.astype(jnp.bfloat16), v_ref[h], (((1,), (0,)), ((), ())),
                preferred_element_type=jnp.float32)
            pv_buf[:, h * DH:(h + 1) * DH] = pv
            loc_ref[0, :, h:h + 1] = m_loc
            loc_ref[1, :, h:h + 1] = l_loc

        @pl.when(hop > 0)
        def _():
            acc_rdma(hop - 1).wait_send()
            st_rdma(hop - 1).wait_send()

            @pl.when(hop - 1 < N_DEV - 1)
            def _():
                q_rdma(hop - 1).wait_send()

        @pl.when(hop > 0)
        def _():
            acc_rdma(hop - 1).wait_recv()
            st_rdma(hop - 1).wait_recv()

        for h in range(HQ):
            m_prev = st_buf[hop, 0, :, h:h + 1]
            l_prev = st_buf[hop, 1, :, h:h + 1]
            m_loc = loc_ref[0, :, h:h + 1]
            l_loc = loc_ref[1, :, h:h + 1]
            m_new = jnp.maximum(m_prev, m_loc)
            alpha = jnp.exp(m_prev - m_new)
            beta = jnp.exp(m_loc - m_new)
            acc = acc_buf[hop, :, h * DH:(h + 1) * DH].astype(jnp.float32)
            pv = pv_buf[:, h * DH:(h + 1) * DH]
            acc_buf[hop, :, h * DH:(h + 1) * DH] = (
                acc * alpha + pv * beta).astype(jnp.bfloat16)
            st_buf[hop, 0, :, h:h + 1] = m_new
            st_buf[hop, 1, :, h:h + 1] = l_prev * alpha + l_loc * beta

        acc_rdma(hop).start()
        st_rdma(hop).start()
        return carry

    lax.fori_loop(0, N_DEV, hop_body, 0)

    acc_rdma(N_DEV - 1).wait_send()
    st_rdma(N_DEV - 1).wait_send()
    acc_rdma(N_DEV - 1).wait_recv()
    st_rdma(N_DEV - 1).wait_recv()

    parts = []
    for h in range(HQ):
        acc = acc_buf[N_DEV, :, h * DH:(h + 1) * DH].astype(jnp.float32)
        l = st_buf[N_DEV, 1, :, h:h + 1]
        parts.append(acc / l)
    ctx = jnp.concatenate(parts, axis=1)
    out_ref[...] = jnp.dot(ctx.astype(jnp.bfloat16), wo_ref[...],
                           preferred_element_type=jnp.float32)


def kernel(x, Wq, K_ext, V_ext, Wo):
    xb = x[0].astype(jnp.bfloat16)
    wq = Wq.astype(jnp.bfloat16)
    wo = Wo.astype(jnp.bfloat16)
    kb = jnp.transpose(K_ext[0].astype(jnp.bfloat16), (1, 0, 2))
    vb = jnp.transpose(V_ext[0].astype(jnp.bfloat16), (1, 0, 2))

    out = pl.pallas_call(
        _body,
        out_shape=jax.ShapeDtypeStruct((SQ, D), jnp.float32),
        in_specs=[pl.BlockSpec(memory_space=pltpu.VMEM)] * 5,
        out_specs=pl.BlockSpec(memory_space=pltpu.VMEM),
        scratch_shapes=[
            pltpu.VMEM((N_DEV + 1, SQ, D), jnp.bfloat16),
            pltpu.VMEM((N_DEV + 1, SQ, D), jnp.bfloat16),
            pltpu.VMEM((N_DEV + 1, 2, SQ, HQ), jnp.float32),
            pltpu.VMEM((SQ, SKV), jnp.float32),
            pltpu.VMEM((SQ, D), jnp.float32),
            pltpu.VMEM((2, SQ, HQ), jnp.float32),
            pltpu.SemaphoreType.DMA((N_DEV + 1,)),
            pltpu.SemaphoreType.DMA((N_DEV + 1,)),
            pltpu.SemaphoreType.DMA((N_DEV + 1,)),
            pltpu.SemaphoreType.DMA((N_DEV + 1,)),
            pltpu.SemaphoreType.DMA((N_DEV + 1,)),
            pltpu.SemaphoreType.DMA((N_DEV + 1,)),
        ],
        compiler_params=pltpu.CompilerParams(
            collective_id=0,
            vmem_limit_bytes=100 * 1024 * 1024,
        ),
    )(xb, wq, kb, vb, wo)
    return out[None]
